# baseline (speedup 1.0000x reference)
import functools

import jax
import jax.numpy as jnp
from jax import lax
from jax.experimental import pallas as pl
from jax.experimental.pallas import tpu as pltpu
from jax.experimental.pallas import tpu_sc as plsc


def _ids_body(z_ref, w_ref, ids_ref):
    w = w_ref[...]
    K = w.shape[0]
    c = jnp.sum(w * w, axis=1)[:, None]
    nb = z_ref.shape[0]
    px = z_ref.shape[2]
    iota = jax.lax.broadcasted_iota(jnp.int32, (K, px), 0)
    for i in range(nb):
        zc = z_ref[i]
        b2 = jax.lax.dot_general(w, zc, (((1,), (0,)), ((), ())),
                                 preferred_element_type=jnp.float32)
        a = jnp.sum(zc * zc, axis=0)[None, :]
        dist = (a - 2.0 * b2) + c
        mval = jnp.min(dist, axis=0, keepdims=True)
        ids2 = jnp.min(jnp.where(dist == mval, iota, K), axis=0,
                       keepdims=True)
        ids_ref[i, 0] = ids2


def _xpose_body(rows_ref, ek_ref):
    nb = rows_ref.shape[0]
    C = ek_ref.shape[1]
    for i in range(nb):
        ek_ref[i] = rows_ref[i][:, :C].T


def _sc_gather(table, idx):
    info = plsc.get_sparse_core_info()
    NC, NS = info.num_cores, info.num_subcores
    NW = NC * NS
    Bp = idx.shape[0]
    b_per_w = Bp // NW
    D = table.shape[1]
    mesh = plsc.VectorSubcoreMesh(core_axis_name="c", subcore_axis_name="s")

    @functools.partial(
        pl.kernel, mesh=mesh,
        out_type=jax.ShapeDtypeStruct((Bp, D), jnp.float32),
        scratch_types=[
            pltpu.VMEM((b_per_w,), jnp.int32),
            pltpu.VMEM((b_per_w, D), jnp.float32),
            pltpu.SemaphoreType.DMA,
        ],
    )
    def k(table_hbm, idx_hbm, out_hbm, idx_v, rows_v, sem):
        wid = lax.axis_index("s") * NC + lax.axis_index("c")
        base = wid * b_per_w
        pltpu.sync_copy(idx_hbm.at[pl.ds(base, b_per_w)], idx_v)
        pltpu.async_copy(table_hbm.at[idx_v], rows_v, sem).wait()
        pltpu.sync_copy(rows_v, out_hbm.at[pl.ds(base, b_per_w)])

    return k(table, idx)


def kernel(z_e, codebook):
    B, C, H, W = z_e.shape
    K = codebook.shape[0]
    P = H * W
    GB = 4
    ids = pl.pallas_call(
        _ids_body,
        grid=(B // GB,),
        in_specs=[
            pl.BlockSpec((GB, C, P), lambda b: (b, 0, 0)),
            pl.BlockSpec((K, C), lambda b: (0, 0)),
        ],
        out_specs=pl.BlockSpec((GB, 1, 1, P), lambda b: (b, 0, 0, 0)),
        out_shape=jax.ShapeDtypeStruct((B, 1, 1, P), jnp.int32),
        compiler_params=pltpu.CompilerParams(
            dimension_semantics=("parallel",)),
    )(z_e.reshape(B, C, P), codebook)

    wpad = jnp.pad(codebook, ((0, 0), (0, 128 - C)))
    rows = _sc_gather(wpad, ids.reshape(B * P))

    ek = pl.pallas_call(
        _xpose_body,
        grid=(B // GB,),
        in_specs=[pl.BlockSpec((GB, P, 128), lambda b: (b, 0, 0))],
        out_specs=pl.BlockSpec((GB, C, P), lambda b: (b, 0, 0)),
        out_shape=jax.ShapeDtypeStruct((B, C, P), jnp.float32),
        compiler_params=pltpu.CompilerParams(
            dimension_semantics=("parallel",)),
    )(rows.reshape(B, P, 128))
    ek = ek.reshape(B, C, H, W)
    return ek, ek, ids.reshape(B, H, W)


# speedup vs baseline: 1.4864x; 1.4864x over previous
import jax
import jax.numpy as jnp
from jax.experimental import pallas as pl
from jax.experimental.pallas import tpu as pltpu


def _vq_body(z_ref, w_ref, ek_ref, ids_ref):
    w = w_ref[...]
    K = w.shape[0]
    c = jnp.sum(w * w, axis=1)[:, None]
    w2 = w + w
    nb = z_ref.shape[0]
    px = z_ref.shape[2]
    iota = jax.lax.broadcasted_iota(jnp.int32, (K, px), 0)
    for i in range(nb):
        zc = z_ref[i]
        b2 = jax.lax.dot_general(w2, zc, (((1,), (0,)), ((), ())),
                                 preferred_element_type=jnp.float32)
        a = jnp.sum(zc * zc, axis=0)[None, :]
        dist = (a - b2) + c
        mval = jnp.min(dist, axis=0, keepdims=True)
        ids2 = jnp.min(jnp.where(dist == mval, iota, K), axis=0,
                       keepdims=True)
        onehot = (iota == ids2).astype(jnp.float32)
        ek = jax.lax.dot_general(w, onehot, (((0,), (0,)), ((), ())),
                                 preferred_element_type=jnp.float32)
        ek_ref[i] = ek
        ids_ref[i, 0] = ids2


def kernel(z_e, codebook):
    B, C, H, W = z_e.shape
    K = codebook.shape[0]
    P = H * W
    GB = 4
    ek, ids = pl.pallas_call(
        _vq_body,
        grid=(B // GB,),
        in_specs=[
            pl.BlockSpec((GB, C, P), lambda b: (b, 0, 0)),
            pl.BlockSpec((K, C), lambda b: (0, 0)),
        ],
        out_specs=[
            pl.BlockSpec((GB, C, P), lambda b: (b, 0, 0)),
            pl.BlockSpec((GB, 1, 1, P), lambda b: (b, 0, 0, 0)),
        ],
        out_shape=[
            jax.ShapeDtypeStruct((B, C, P), jnp.float32),
            jax.ShapeDtypeStruct((B, 1, 1, P), jnp.int32),
        ],
        compiler_params=pltpu.CompilerParams(
            dimension_semantics=("parallel",)),
    )(z_e.reshape(B, C, P), codebook)
    ek = ek.reshape(B, C, H, W)
    return ek, ek, ids.reshape(B, H, W)
